# baseline (speedup 1.0000x reference)
import functools
import jax
import jax.numpy as jnp
from jax import lax
from jax.experimental import pallas as pl
from jax.experimental.pallas import tpu as pltpu, tpu_sc as plsc

B, S, D = 4, 8192, 768
NW = 32
ROWS_PER_W = S // NW
CHUNK_ROWS = 32
HALF = CHUNK_ROWS // 2
N_CHUNKS = ROWS_PER_W // CHUNK_ROWS
NBUF = 3
PREF = 1
STEPS = [(c, b) for c in range(N_CHUNKS) for b in range(B)]
NSTEPS = len(STEPS)


def _sc_body(x_hbm, t_hbm, o_hbm, *refs):
    xbufs = list(refs[0:NBUF])
    tbufs = list(refs[NBUF:NBUF + 2])
    xsems = list(refs[NBUF + 2:2 * NBUF + 2])
    tsems = list(refs[2 * NBUF + 2:2 * NBUF + 4])
    osems = list(refs[2 * NBUF + 4:3 * NBUF + 4])
    wid = lax.axis_index("s") * 2 + lax.axis_index("c")
    base = wid * ROWS_PER_W

    def row0(c):
        return base + c * CHUNK_ROWS

    tdesc = [None] * N_CHUNKS
    xdesc = [None] * NSTEPS
    odesc = [[] for _ in range(NSTEPS)]

    tdesc[0] = pltpu.async_copy(t_hbm.at[pl.ds(row0(0), CHUNK_ROWS)],
                                tbufs[0], tsems[0])
    for j in range(min(PREF, NSTEPS)):
        cj, bj = STEPS[j]
        xdesc[j] = pltpu.async_copy(x_hbm.at[bj, pl.ds(row0(cj), CHUNK_ROWS)],
                                    xbufs[j % NBUF], xsems[j % NBUF])

    for k, (c, b) in enumerate(STEPS):
        if b == 0:
            tdesc[c].wait()
            if c + 1 < N_CHUNKS:
                tdesc[c + 1] = pltpu.async_copy(
                    t_hbm.at[pl.ds(row0(c + 1), CHUNK_ROWS)],
                    tbufs[(c + 1) % 2], tsems[(c + 1) % 2])
        xdesc[k].wait()
        j = k + PREF
        if j < NSTEPS:
            jj = j - NBUF
            if jj >= 0:
                for d in odesc[jj]:
                    d.wait()
            cj, bj = STEPS[j]
            xdesc[j] = pltpu.async_copy(x_hbm.at[bj, pl.ds(row0(cj), CHUNK_ROWS)],
                                        xbufs[j % NBUF], xsems[j % NBUF])
        xbuf, tbuf = xbufs[k % NBUF], tbufs[c % 2]

        for h in range(2):
            r_lo = h * HALF

            @plsc.parallel_loop(r_lo, r_lo + HALF, 1)
            def _add(r):
                @plsc.parallel_loop(0, D, 16, unroll=8)
                def _add_row(s0):
                    plsc.addupdate(xbuf.at[r, pl.ds(s0, 16)],
                                   tbuf[r, pl.ds(s0, 16)])

            odesc[k].append(pltpu.async_copy(
                xbuf.at[pl.ds(r_lo, HALF)],
                o_hbm.at[b, pl.ds(row0(c) + r_lo, HALF)],
                osems[k % NBUF]))
    for k in range(max(0, NSTEPS - NBUF), NSTEPS):
        for d in odesc[k]:
            d.wait()


def kernel(x, pos_table):
    mesh = plsc.VectorSubcoreMesh(core_axis_name="c", subcore_axis_name="s")
    k = functools.partial(
        pl.kernel,
        out_type=jax.ShapeDtypeStruct((B, S, D), jnp.float32),
        mesh=mesh,
        scratch_types=(
            [pltpu.VMEM((CHUNK_ROWS, D), jnp.float32)] * (NBUF + 2)
            + [pltpu.SemaphoreType.DMA] * (2 * NBUF + 2)
        ),
    )(_sc_body)
    return k(x, pos_table)


# speedup vs baseline: 1.0551x; 1.0551x over previous
import functools
import jax
import jax.numpy as jnp
from jax import lax
from jax.experimental import pallas as pl
from jax.experimental.pallas import tpu as pltpu, tpu_sc as plsc

B, S, D = 4, 8192, 768
NW = 32
ROWS_PER_W = S // NW
CHUNK_ROWS = 8
N_CHUNKS = ROWS_PER_W // CHUNK_ROWS
NGEN = 3
NXB = NGEN * B


def _sc_body(x_hbm, t_hbm, o_hbm, *refs):
    xbufs = list(refs[0:NXB])
    tbufs = list(refs[NXB:NXB + 2])
    xsems = list(refs[NXB + 2:2 * NXB + 2])
    tsems = list(refs[2 * NXB + 2:2 * NXB + 4])
    osems = list(refs[2 * NXB + 4:3 * NXB + 4])
    wid = lax.axis_index("s") * 2 + lax.axis_index("c")
    base = wid * ROWS_PER_W

    def row0(c):
        return base + c * CHUNK_ROWS

    tdesc = [None] * N_CHUNKS
    xdesc = [[None] * B for _ in range(N_CHUNKS)]
    odesc = [[None] * B for _ in range(N_CHUNKS)]

    def issue_gathers(c):
        g = c % NGEN
        for b in range(B):
            xdesc[c][b] = pltpu.async_copy(
                x_hbm.at[b, pl.ds(row0(c), CHUNK_ROWS)],
                xbufs[4 * g + b], xsems[4 * g + b])

    tdesc[0] = pltpu.async_copy(t_hbm.at[pl.ds(row0(0), CHUNK_ROWS)],
                                tbufs[0], tsems[0])
    issue_gathers(0)
    issue_gathers(1)

    for c in range(N_CHUNKS):
        tdesc[c].wait()
        if c + 1 < N_CHUNKS:
            tdesc[c + 1] = pltpu.async_copy(
                t_hbm.at[pl.ds(row0(c + 1), CHUNK_ROWS)],
                tbufs[(c + 1) % 2], tsems[(c + 1) % 2])
        for b in range(B):
            xdesc[c][b].wait()
        if c + 2 < N_CHUNKS:
            if c - 1 >= 0:
                for b in range(B):
                    odesc[c - 1][b].wait()
            issue_gathers(c + 2)

        g = c % NGEN
        tbuf = tbufs[c % 2]
        bb = [xbufs[4 * g + b] for b in range(B)]

        @plsc.parallel_loop(0, CHUNK_ROWS, 1)
        def _add(r):
            @plsc.parallel_loop(0, D, 16, unroll=4)
            def _add_row(s0):
                v = tbuf[r, pl.ds(s0, 16)]
                plsc.addupdate(bb[0].at[r, pl.ds(s0, 16)], v)
                plsc.addupdate(bb[1].at[r, pl.ds(s0, 16)], v)
                plsc.addupdate(bb[2].at[r, pl.ds(s0, 16)], v)
                plsc.addupdate(bb[3].at[r, pl.ds(s0, 16)], v)

        for b in range(B):
            odesc[c][b] = pltpu.async_copy(
                xbufs[4 * g + b],
                o_hbm.at[b, pl.ds(row0(c), CHUNK_ROWS)],
                osems[4 * g + b])
    for c in range(N_CHUNKS - 2, N_CHUNKS):
        for b in range(B):
            odesc[c][b].wait()


def kernel(x, pos_table):
    mesh = plsc.VectorSubcoreMesh(core_axis_name="c", subcore_axis_name="s")
    k = functools.partial(
        pl.kernel,
        out_type=jax.ShapeDtypeStruct((B, S, D), jnp.float32),
        mesh=mesh,
        scratch_types=(
            [pltpu.VMEM((CHUNK_ROWS, D), jnp.float32)] * (NXB + 2)
            + [pltpu.SemaphoreType.DMA] * (2 * NXB + 2)
        ),
    )(_sc_body)
    return k(x, pos_table)
